# Initial kernel scaffold; baseline (speedup 1.0000x reference)
import functools

import jax
import jax.numpy as jnp
from jax import lax
from jax.experimental import pallas as pl
from jax.experimental.pallas import tpu as pltpu
from jax.experimental.pallas import tpu_sc as plsc

N = 2048
E = 65536
B = 8
NPG = 256
H = 64
K7 = 7
TI = 64
RT = TI + 6
NT = NPG // TI

_MM_PREC = lax.Precision.HIGH


NW = 32
RPR = 32
NROUNDS = N // (NW * RPR)
CHUNK = 8192
NCHUNK = E // CHUNK
ACC = RPR * N


def _build_adj(src, dst, w):
    mesh = plsc.VectorSubcoreMesh(core_axis_name="c", subcore_axis_name="s")

    @functools.partial(
        pl.kernel,
        out_type=jax.ShapeDtypeStruct((N * N,), jnp.float32),
        mesh=mesh,
        scratch_types=[
            pltpu.VMEM((CHUNK,), jnp.int32),
            pltpu.VMEM((CHUNK,), jnp.int32),
            pltpu.VMEM((CHUNK,), jnp.float32),
            pltpu.VMEM((ACC,), jnp.float32),
        ],
    )
    def k(src_hbm, dst_hbm, w_hbm, out_hbm, s_v, d_v, w_v, acc_v):
        wid = lax.axis_index("s") * 2 + lax.axis_index("c")

        @pl.loop(0, NROUNDS)
        def _round(r):
            row0 = (r * NW + wid) * RPR

            @pl.loop(0, ACC // 64)
            def _zero(i):
                z = jnp.zeros((16,), jnp.float32)
                acc_v[pl.ds(i * 64, 16)] = z
                acc_v[pl.ds(i * 64 + 16, 16)] = z
                acc_v[pl.ds(i * 64 + 32, 16)] = z
                acc_v[pl.ds(i * 64 + 48, 16)] = z

            @pl.loop(0, NCHUNK)
            def _chunk(c):
                e0 = c * CHUNK
                pltpu.sync_copy(src_hbm.at[pl.ds(e0, CHUNK)], s_v)
                pltpu.sync_copy(dst_hbm.at[pl.ds(e0, CHUNK)], d_v)
                pltpu.sync_copy(w_hbm.at[pl.ds(e0, CHUNK)], w_v)

                @pl.loop(0, CHUNK // 16)
                def _scan(i):
                    sl = pl.ds(i * 16, 16)
                    rel = d_v[sl] - row0
                    m = (rel >= 0) & (rel < RPR)
                    idx = jnp.where(m, rel * N + s_v[sl], 0)
                    plsc.addupdate_scatter(acc_v, [idx], w_v[sl], mask=m)

            pltpu.sync_copy(acc_v, out_hbm.at[pl.ds(row0 * N, ACC)])

    return k(src, dst, w)


def _gcn_body(a_ref, x_ref, w0_ref, b0_ref, w1_ref, b1_ref, w2_ref, b2_ref,
              out_ref):
    a = a_ref[...]
    deg = jnp.sum(a, axis=1) + 2.0
    dinv = jnp.where(deg > 0, lax.rsqrt(deg), 0.0)
    dcol = dinv[:, None]
    loop_w = 2.0 * dcol * dcol
    h = x_ref[...]
    for w_r, b_r in ((w0_ref, b0_ref), (w1_ref, b1_ref), (w2_ref, b2_ref)):
        hp = jnp.dot(h, w_r[...], precision=_MM_PREC)
        u = jnp.dot(a, hp * dcol, precision=_MM_PREC)
        h = jax.nn.relu(dcol * u + loop_w * hp + b_r[...][None, :])
    out_ref[...] = h.T


def _gcn(a, x, w0, b0, w1, b1, w2, b2):
    return pl.pallas_call(
        _gcn_body,
        out_shape=jax.ShapeDtypeStruct((H, N), jnp.float32),
    )(a, x, w0, b0, w1, b1, w2, b2)


def _decode_body(z_ref, whd_ref, whm_ref, wp_ref, bh_ref, bp_ref, out_ref):
    t = pl.program_id(1)
    i0 = t * TI
    st = jnp.clip(i0 - 3, 0, NPG - RT)
    off = i0 - st + 3

    zt = z_ref[...]
    zrows = lax.dynamic_slice(zt, (0, st), (H, RT))
    diff = jnp.abs(zrows[:, :, None] - zt[:, None, :])
    prod = zrows[:, :, None] * zt[:, None, :]
    d2 = diff.reshape(H, RT * NPG)
    p2 = prod.reshape(H, RT * NPG)
    hmt = jax.nn.relu(
        jnp.dot(whd_ref[...], d2, precision=_MM_PREC)
        + jnp.dot(whm_ref[...], p2, precision=_MM_PREC)
        + bh_ref[...][:, None])
    g = jnp.dot(wp_ref[...], hmt, precision=_MM_PREC)
    gp = jnp.pad(g.reshape(K7 * K7, RT, NPG),
                 ((0, 0), (3, 3), (3, 3)))

    y1 = jnp.zeros((TI, NPG), jnp.float32)
    y2 = jnp.zeros((TI, NPG), jnp.float32)
    for p in range(K7):
        for q in range(K7):
            r0 = off + p - 3
            s1 = lax.dynamic_slice(gp, (p * K7 + q, r0, q), (1, TI, NPG))
            y1 = y1 + s1[0]
            s2 = lax.dynamic_slice(gp, (q * K7 + p, r0, q), (1, TI, NPG))
            y2 = y2 + s2[0]
    bp = bp_ref[0, 0]
    c = 0.5 * (jax.nn.sigmoid(y1 + bp) + jax.nn.sigmoid(y2 + bp))
    out_ref[0, 0] = c


def _decode(h3t, whd, whm, wp49, bh, bp2):
    return pl.pallas_call(
        _decode_body,
        grid=(B, NT),
        in_specs=[
            pl.BlockSpec((H, NPG), lambda b, t: (0, b)),
            pl.BlockSpec((H, H), lambda b, t: (0, 0)),
            pl.BlockSpec((H, H), lambda b, t: (0, 0)),
            pl.BlockSpec((K7 * K7, H), lambda b, t: (0, 0)),
            pl.BlockSpec((H,), lambda b, t: (0,)),
            pl.BlockSpec((1, 1), lambda b, t: (0, 0)),
        ],
        out_specs=pl.BlockSpec((1, 1, TI, NPG), lambda b, t: (b, 0, t, 0)),
        out_shape=jax.ShapeDtypeStruct((B, 1, NPG, NPG), jnp.float32),
    )(h3t, whd, whm, wp49, bh, bp2)


def kernel(x, edge_index, edge_attr, batch, W0, b0, W1, b1, W2, b2,
           Wh, bh, Wp, bp):
    src = edge_index[0]
    dst = edge_index[1]
    a = _build_adj(src, dst, edge_attr).reshape(N, N)
    h3t = _gcn(a, x, W0, b0, W1, b1, W2, b2)
    whd = Wh[:, :H, 0, 0]
    whm = Wh[:, H:, 0, 0]
    wp49 = Wp[0].reshape(H, K7 * K7).T
    bp2 = bp.reshape(1, 1)
    return _decode(h3t, whd, whm, wp49, bh, bp2)


# speedup vs baseline: 9.7487x; 9.7487x over previous
import dataclasses
import functools

import jax
import jax.numpy as jnp
from jax import lax
from jax.experimental import pallas as pl
from jax.experimental.pallas import tpu as pltpu
from jax.experimental.pallas import tpu_sc as plsc

N = 2048
E = 65536
B = 8
NPG = 256
H = 64
K7 = 7
TI = 64
RT = 80
NT = NPG // TI
GPR = 96

_MM_PREC = lax.Precision.DEFAULT


NW = 32
RPR = 32
NROUNDS = N // (NW * RPR)
CHUNK = 8192
NCHUNK = E // CHUNK
ACC = RPR * N


_SC_PARAMS = pltpu.CompilerParams()
if "needs_layout_passes" in pltpu.CompilerParams.__dataclass_fields__:
    _SC_PARAMS = dataclasses.replace(_SC_PARAMS, needs_layout_passes=False)


def _build_adj(src, dst, w):
    mesh = plsc.VectorSubcoreMesh(core_axis_name="c", subcore_axis_name="s")

    @functools.partial(
        pl.kernel,
        out_type=jax.ShapeDtypeStruct((N * N,), jnp.float32),
        mesh=mesh,
        compiler_params=_SC_PARAMS,
        scratch_types=[
            pltpu.VMEM((CHUNK,), jnp.int32),
            pltpu.VMEM((CHUNK,), jnp.int32),
            pltpu.VMEM((CHUNK,), jnp.float32),
            pltpu.VMEM((ACC,), jnp.float32),
        ],
    )
    def k(src_hbm, dst_hbm, w_hbm, out_hbm, s_v, d_v, w_v, acc_v):
        wid = lax.axis_index("s") * 2 + lax.axis_index("c")

        @pl.loop(0, NROUNDS)
        def _round(r):
            row0 = (r * NW + wid) * RPR

            @pl.loop(0, ACC // 64)
            def _zero(i):
                z = jnp.zeros((16,), jnp.float32)
                acc_v[pl.ds(i * 64, 16)] = z
                acc_v[pl.ds(i * 64 + 16, 16)] = z
                acc_v[pl.ds(i * 64 + 32, 16)] = z
                acc_v[pl.ds(i * 64 + 48, 16)] = z

            @pl.loop(0, NCHUNK)
            def _chunk(c):
                e0 = c * CHUNK
                pltpu.sync_copy(src_hbm.at[pl.ds(e0, CHUNK)], s_v)
                pltpu.sync_copy(dst_hbm.at[pl.ds(e0, CHUNK)], d_v)
                pltpu.sync_copy(w_hbm.at[pl.ds(e0, CHUNK)], w_v)

                @pl.loop(0, CHUNK // 16)
                def _scan(i):
                    sl = pl.ds(i * 16, 16)
                    rel = d_v[sl] - row0
                    m = (rel >= 0) & (rel < RPR)
                    idx = jnp.where(m, rel * N + s_v[sl], 0)
                    plsc.addupdate_scatter(acc_v, [idx], w_v[sl], mask=m)

            pltpu.sync_copy(acc_v, out_hbm.at[pl.ds(row0 * N, ACC)])

    return k(src, dst, w)


def _gcn_body(a_ref, x_ref, w0_ref, b0_ref, w1_ref, b1_ref, w2_ref, b2_ref,
              out_ref):
    a = a_ref[...]
    deg = jnp.sum(a, axis=1) + 2.0
    dinv = jnp.where(deg > 0, lax.rsqrt(deg), 0.0)
    dcol = dinv[:, None]
    loop_w = 2.0 * dcol * dcol
    h = x_ref[...]
    for w_r, b_r in ((w0_ref, b0_ref), (w1_ref, b1_ref), (w2_ref, b2_ref)):
        hp = jnp.dot(h, w_r[...], precision=_MM_PREC)
        u = jnp.dot(a, hp * dcol, precision=_MM_PREC)
        h = jax.nn.relu(dcol * u + loop_w * hp + b_r[...][None, :])
    out_ref[...] = h


def _gcn(a, x, w0, b0, w1, b1, w2, b2):
    return pl.pallas_call(
        _gcn_body,
        out_shape=jax.ShapeDtypeStruct((N, H), jnp.float32),
    )(a, x, w0, b0, w1, b1, w2, b2)


def _decode_body(z_ref, whd_ref, whm_ref, wp_ref, bh_ref, bp_ref, out_ref,
                 gp_ref):
    t = pl.program_id(1)
    i0 = t * TI
    st = pl.multiple_of(jnp.clip(i0 - 8, 0, NPG - RT), 8)
    woff = pl.multiple_of(16 - (i0 - st), 8)

    zfull = z_ref[...]
    zr = z_ref[pl.ds(st, RT), :]
    zt = zfull.T
    zrt = zr.T
    diff = jnp.abs(zrt[:, :, None] - zt[:, None, :])
    prod = zrt[:, :, None] * zt[:, None, :]
    d2 = diff.reshape(H, RT * NPG)
    p2 = prod.reshape(H, RT * NPG)
    hmt = jax.nn.relu(
        jnp.dot(whd_ref[...], d2, precision=_MM_PREC)
        + jnp.dot(whm_ref[...], p2, precision=_MM_PREC)
        + bh_ref[...][:, None])
    g = jnp.dot(wp_ref[...], hmt, precision=_MM_PREC)
    g3 = g.reshape(K7 * K7, RT, NPG)
    zc = jnp.zeros((K7 * K7, RT, 3), jnp.float32)
    gp_ref[...] = jnp.zeros_like(gp_ref)
    gp_ref[:, pl.ds(woff, RT), :] = jnp.concatenate([zc, g3, zc], axis=2)

    y1 = jnp.zeros((TI, NPG), jnp.float32)
    y2 = jnp.zeros((TI, NPG), jnp.float32)
    for p in range(K7):
        for q in range(K7):
            y1 = y1 + gp_ref[p * K7 + q, pl.ds(13 + p, TI), pl.ds(q, NPG)]
            y2 = y2 + gp_ref[q * K7 + p, pl.ds(13 + p, TI), pl.ds(q, NPG)]
    bp = bp_ref[0, 0]
    c = 0.5 * (jax.nn.sigmoid(y1 + bp) + jax.nn.sigmoid(y2 + bp))
    out_ref[0, 0] = c


def _decode(h3, whd, whm, wp49, bh, bp2):
    return pl.pallas_call(
        _decode_body,
        grid=(B, NT),
        in_specs=[
            pl.BlockSpec((NPG, H), lambda b, t: (b, 0)),
            pl.BlockSpec((H, H), lambda b, t: (0, 0)),
            pl.BlockSpec((H, H), lambda b, t: (0, 0)),
            pl.BlockSpec((K7 * K7, H), lambda b, t: (0, 0)),
            pl.BlockSpec((H,), lambda b, t: (0,)),
            pl.BlockSpec((1, 1), lambda b, t: (0, 0)),
        ],
        out_specs=pl.BlockSpec((1, 1, TI, NPG), lambda b, t: (b, 0, t, 0)),
        out_shape=jax.ShapeDtypeStruct((B, 1, NPG, NPG), jnp.float32),
        scratch_shapes=[pltpu.VMEM((K7 * K7, GPR, NPG + 6), jnp.float32)],
    )(h3, whd, whm, wp49, bh, bp2)


def kernel(x, edge_index, edge_attr, batch, W0, b0, W1, b1, W2, b2,
           Wh, bh, Wp, bp):
    src = edge_index[0]
    dst = edge_index[1]
    a = _build_adj(src, dst, edge_attr).reshape(N, N)
    h3 = _gcn(a, x, W0, b0, W1, b1, W2, b2)
    whd = Wh[:, :H, 0, 0]
    whm = Wh[:, H:, 0, 0]
    wp49 = Wp[0].reshape(H, K7 * K7).T
    bp2 = bp.reshape(1, 1)
    return _decode(h3, whd, whm, wp49, bh, bp2)


# speedup vs baseline: 11.2794x; 1.1570x over previous
import dataclasses
import functools

import jax
import jax.numpy as jnp
from jax import lax
from jax.experimental import pallas as pl
from jax.experimental.pallas import tpu as pltpu
from jax.experimental.pallas import tpu_sc as plsc

N = 2048
E = 65536
B = 8
NPG = 256
H = 64
K7 = 7
TI = 64
RT = 80
NT = NPG // TI
GPR = 96

_MM_PREC = lax.Precision.DEFAULT


NW = 32
RPR = 32
NROUNDS = N // (NW * RPR)
CHUNK = 16384
NCHUNK = E // CHUNK
ACC = RPR * N


_SC_PARAMS = pltpu.CompilerParams()
if "needs_layout_passes" in pltpu.CompilerParams.__dataclass_fields__:
    _SC_PARAMS = dataclasses.replace(_SC_PARAMS, needs_layout_passes=False)


def _build_adj(src, dst, w):
    mesh = plsc.VectorSubcoreMesh(core_axis_name="c", subcore_axis_name="s")

    @functools.partial(
        pl.kernel,
        out_type=jax.ShapeDtypeStruct((N * N,), jnp.float32),
        mesh=mesh,
        compiler_params=_SC_PARAMS,
        scratch_types=[
            pltpu.VMEM((CHUNK,), jnp.int32),
            pltpu.VMEM((CHUNK,), jnp.int32),
            pltpu.VMEM((CHUNK,), jnp.float32),
            pltpu.VMEM((ACC,), jnp.float32),
        ],
    )
    def k(src_hbm, dst_hbm, w_hbm, out_hbm, s_v, d_v, w_v, acc_v):
        wid = lax.axis_index("s") * 2 + lax.axis_index("c")

        @pl.loop(0, NROUNDS)
        def _round(r):
            row0 = (r * NW + wid) * RPR

            @pl.loop(0, ACC // 64)
            def _zero(i):
                z = jnp.zeros((16,), jnp.float32)
                acc_v[pl.ds(i * 64, 16)] = z
                acc_v[pl.ds(i * 64 + 16, 16)] = z
                acc_v[pl.ds(i * 64 + 32, 16)] = z
                acc_v[pl.ds(i * 64 + 48, 16)] = z

            @pl.loop(0, NCHUNK)
            def _chunk(c):
                e0 = c * CHUNK
                pltpu.sync_copy(src_hbm.at[pl.ds(e0, CHUNK)], s_v)
                pltpu.sync_copy(dst_hbm.at[pl.ds(e0, CHUNK)], d_v)
                pltpu.sync_copy(w_hbm.at[pl.ds(e0, CHUNK)], w_v)

                @pl.loop(0, CHUNK // 64)
                def _scan(i):
                    for u in range(4):
                        sl = pl.ds(i * 64 + u * 16, 16)
                        rel = d_v[sl] - row0
                        m = (rel >= 0) & (rel < RPR)
                        idx = jnp.where(m, rel * N + s_v[sl], 0)
                        plsc.addupdate_scatter(acc_v, [idx], w_v[sl], mask=m)

            pltpu.sync_copy(acc_v, out_hbm.at[pl.ds(row0 * N, ACC)])

    return k(src, dst, w)


def _gcn_body(a_ref, x_ref, w0_ref, b0_ref, w1_ref, b1_ref, w2_ref, b2_ref,
              out_ref, outt_ref):
    a = a_ref[...]
    deg = jnp.sum(a, axis=1) + 2.0
    dinv = jnp.where(deg > 0, lax.rsqrt(deg), 0.0)
    dcol = dinv[:, None]
    loop_w = 2.0 * dcol * dcol
    h = x_ref[...]
    for w_r, b_r in ((w0_ref, b0_ref), (w1_ref, b1_ref), (w2_ref, b2_ref)):
        hp = jnp.dot(h, w_r[...], precision=_MM_PREC)
        u = jnp.dot(a, hp * dcol, precision=_MM_PREC)
        h = jax.nn.relu(dcol * u + loop_w * hp + b_r[...][None, :])
    out_ref[...] = h
    outt_ref[...] = h.T.astype(jnp.bfloat16)


def _gcn(a, x, w0, b0, w1, b1, w2, b2):
    return pl.pallas_call(
        _gcn_body,
        out_shape=(jax.ShapeDtypeStruct((N, H), jnp.float32),
                   jax.ShapeDtypeStruct((H, N), jnp.bfloat16)),
    )(a, x, w0, b0, w1, b1, w2, b2)


def _decode_body(z_ref, zt_ref, whc_ref, wp_ref, bh_ref, bp_ref, out_ref,
                 gp_ref):
    t = pl.program_id(1)
    i0 = t * TI
    st = pl.multiple_of(jnp.clip(i0 - 8, 0, NPG - RT), 8)
    woff = pl.multiple_of(16 - (i0 - st), 8)

    zt = zt_ref[...]
    zr = z_ref[pl.ds(st, RT), :]
    zrt = zr.T.astype(jnp.bfloat16)
    diff = jnp.abs(zrt[:, :, None] - zt[:, None, :])
    prod = zrt[:, :, None] * zt[:, None, :]
    cat = jnp.concatenate([diff, prod], axis=0)
    c2 = cat.reshape(2 * H, RT * NPG)
    hmt = jax.nn.relu(
        jnp.dot(whc_ref[...], c2, precision=_MM_PREC,
                preferred_element_type=jnp.float32)
        + bh_ref[...][:, None]).astype(jnp.bfloat16)
    g = jnp.dot(wp_ref[...], hmt, precision=_MM_PREC,
                preferred_element_type=jnp.float32)
    g3 = g.reshape(K7 * K7, RT, NPG)
    zc = jnp.zeros((K7 * K7, RT, 3), jnp.float32)
    gp_ref[:, pl.ds(8, 8), :] = jnp.zeros((K7 * K7, 8, NPG + 6), jnp.float32)
    gp_ref[:, pl.ds(80, 8), :] = jnp.zeros((K7 * K7, 8, NPG + 6), jnp.float32)
    gp_ref[:, pl.ds(woff, RT), :] = jnp.concatenate([zc, g3, zc], axis=2)

    y1 = jnp.zeros((TI, NPG), jnp.float32)
    y2 = jnp.zeros((TI, NPG), jnp.float32)
    for a in range(K7):
        for b in range(K7):
            slab = gp_ref[a * K7 + b, pl.ds(13, 70), :]
            y1 = y1 + lax.slice(slab, (a, b), (a + TI, b + NPG))
            y2 = y2 + lax.slice(slab, (b, a), (b + TI, a + NPG))
    bp = bp_ref[0, 0]
    c = 0.5 * (jax.nn.sigmoid(y1 + bp) + jax.nn.sigmoid(y2 + bp))
    out_ref[0, 0] = c


def _decode(h3, h3t, whc, wp49, bh, bp2):
    return pl.pallas_call(
        _decode_body,
        grid=(B, NT),
        in_specs=[
            pl.BlockSpec((NPG, H), lambda b, t: (b, 0)),
            pl.BlockSpec((H, NPG), lambda b, t: (0, b)),
            pl.BlockSpec((H, 2 * H), lambda b, t: (0, 0)),
            pl.BlockSpec((K7 * K7, H), lambda b, t: (0, 0)),
            pl.BlockSpec((H,), lambda b, t: (0,)),
            pl.BlockSpec((1, 1), lambda b, t: (0, 0)),
        ],
        out_specs=pl.BlockSpec((1, 1, TI, NPG), lambda b, t: (b, 0, t, 0)),
        out_shape=jax.ShapeDtypeStruct((B, 1, NPG, NPG), jnp.float32),
        scratch_shapes=[pltpu.VMEM((K7 * K7, GPR, NPG + 6), jnp.float32)],
    )(h3, h3t, whc, wp49, bh, bp2)


def kernel(x, edge_index, edge_attr, batch, W0, b0, W1, b1, W2, b2,
           Wh, bh, Wp, bp):
    src = edge_index[0]
    dst = edge_index[1]
    a = _build_adj(src, dst, edge_attr).reshape(N, N)
    h3, h3t = _gcn(a, x, W0, b0, W1, b1, W2, b2)
    whc = Wh[:, :, 0, 0].astype(jnp.bfloat16)
    wp49 = Wp[0].reshape(H, K7 * K7).T.astype(jnp.bfloat16)
    bp2 = bp.reshape(1, 1)
    return _decode(h3, h3t, whc, wp49, bh, bp2)


# speedup vs baseline: 14.5240x; 1.2877x over previous
import dataclasses
import functools

import jax
import jax.numpy as jnp
from jax import lax
from jax.experimental import pallas as pl
from jax.experimental.pallas import tpu as pltpu
from jax.experimental.pallas import tpu_sc as plsc

N = 2048
E = 65536
B = 8
NPG = 256
H = 64
K7 = 7
TI = 64
RT = 80
NT = NPG // TI
GPR = 96

_MM_PREC = lax.Precision.DEFAULT


SUB = 16
EPT = E // SUB
SROWS = 512
TRASH = SROWS * N
SHW = TRASH + 128
ZBLK = 16384
SLICE = SROWS * N // SUB


_SC_PARAMS = pltpu.CompilerParams()
if "needs_layout_passes" in pltpu.CompilerParams.__dataclass_fields__:
    _SC_PARAMS = dataclasses.replace(_SC_PARAMS, needs_layout_passes=False)


def _build_adj(src, dst, w):
    mesh = plsc.VectorSubcoreMesh(core_axis_name="c", subcore_axis_name="s")

    @functools.partial(
        pl.kernel,
        out_type=jax.ShapeDtypeStruct((N * N,), jnp.float32),
        mesh=mesh,
        compiler_params=_SC_PARAMS,
        scratch_types=[
            pltpu.VMEM((EPT,), jnp.int32),
            pltpu.VMEM((EPT,), jnp.int32),
            pltpu.VMEM((EPT // 128, 128), jnp.float32),
            pltpu.VMEM((EPT // 128, 128), jnp.int32),
            pltpu.VMEM((ZBLK,), jnp.float32),
            pltpu.VMEM_SHARED((SHW,), jnp.float32),
            pltpu.SemaphoreType.DMA,
        ],
    )
    def k(src_hbm, dst_hbm, w_hbm, out_hbm, s_v, d_v, w2, idx2, zb, shared,
          sem):
        cid = lax.axis_index("c")
        sid = lax.axis_index("s")
        base_e = sid * EPT
        pltpu.sync_copy(src_hbm.at[pl.ds(base_e, EPT)], s_v)
        pltpu.sync_copy(dst_hbm.at[pl.ds(base_e, EPT)], d_v)
        for j in range(EPT // 128):
            pltpu.async_copy(w_hbm.at[pl.ds(base_e + j * 128, 128)],
                             w2.at[j], sem)

        @pl.loop(0, ZBLK // 64)
        def _zb(i):
            z = jnp.zeros((16,), jnp.float32)
            for u in range(4):
                zb[pl.ds(i * 64 + u * 16, 16)] = z

        for j in range(EPT // 128):
            pltpu.make_async_copy(w_hbm.at[pl.ds(base_e + j * 128, 128)],
                                  w2.at[j], sem).wait()

        lanes = lax.iota(jnp.int32, 16)
        for gi in range(2):
            row0 = (cid * 2 + gi) * SROWS
            zoff = pl.multiple_of(sid * SLICE, 64)
            for j in range(SLICE // ZBLK):
                pltpu.async_copy(zb, shared.at[pl.ds(zoff + j * ZBLK, ZBLK)],
                                 sem)
            for j in range(SLICE // ZBLK):
                pltpu.make_async_copy(
                    zb, shared.at[pl.ds(zoff + j * ZBLK, ZBLK)], sem).wait()
            plsc.subcore_barrier()

            @pl.loop(0, EPT // 128)
            def _mkidx(jj):
                for u in range(8):
                    i = jj * 8 + u
                    sl = pl.ds(i * 16, 16)
                    rel = d_v[sl] - row0
                    m = (rel >= 0) & (rel < SROWS)
                    spread = (TRASH + (i & 7) * 16) + lanes
                    idx = jnp.where(m, rel * N + s_v[sl], spread)
                    idx2[jj, pl.ds(u * 16, 16)] = idx

            for j in range(EPT // 128):
                pltpu.sync_copy(w2.at[j], shared.at[idx2.at[j]], add=True)
            plsc.subcore_barrier()

            out0 = pl.multiple_of(row0 * N + sid * SLICE, 64)
            pltpu.sync_copy(shared.at[pl.ds(zoff, SLICE)],
                            out_hbm.at[pl.ds(out0, SLICE)])
            plsc.subcore_barrier()

    return k(src, dst, w)


def _gcn_body(a_ref, x_ref, w0_ref, b0_ref, w1_ref, b1_ref, w2_ref, b2_ref,
              out_ref, outt_ref):
    a = a_ref[...]
    deg = jnp.sum(a, axis=1) + 2.0
    dinv = jnp.where(deg > 0, lax.rsqrt(deg), 0.0)
    dcol = dinv[:, None]
    loop_w = 2.0 * dcol * dcol
    h = x_ref[...]
    for w_r, b_r in ((w0_ref, b0_ref), (w1_ref, b1_ref), (w2_ref, b2_ref)):
        hp = jnp.dot(h, w_r[...], precision=_MM_PREC)
        u = jnp.dot(a, hp * dcol, precision=_MM_PREC)
        h = jax.nn.relu(dcol * u + loop_w * hp + b_r[...][None, :])
    out_ref[...] = h
    outt_ref[...] = h.T.astype(jnp.bfloat16)


def _gcn(a, x, w0, b0, w1, b1, w2, b2):
    return pl.pallas_call(
        _gcn_body,
        out_shape=(jax.ShapeDtypeStruct((N, H), jnp.float32),
                   jax.ShapeDtypeStruct((H, N), jnp.bfloat16)),
    )(a, x, w0, b0, w1, b1, w2, b2)


def _decode_body(z_ref, zt_ref, whc_ref, wp_ref, bh_ref, bp_ref, out_ref,
                 gp_ref):
    t = pl.program_id(1)
    i0 = t * TI
    st = pl.multiple_of(jnp.clip(i0 - 8, 0, NPG - RT), 8)
    woff = pl.multiple_of(16 - (i0 - st), 8)

    zt = zt_ref[...]
    zr = z_ref[pl.ds(st, RT), :]
    zrt = zr.T.astype(jnp.bfloat16)
    diff = jnp.abs(zrt[:, :, None] - zt[:, None, :])
    prod = zrt[:, :, None] * zt[:, None, :]
    cat = jnp.concatenate([diff, prod], axis=0)
    c2 = cat.reshape(2 * H, RT * NPG)
    hmt = jax.nn.relu(
        jnp.dot(whc_ref[...], c2, precision=_MM_PREC,
                preferred_element_type=jnp.float32)
        + bh_ref[...][:, None]).astype(jnp.bfloat16)
    g = jnp.dot(wp_ref[...], hmt, precision=_MM_PREC,
                preferred_element_type=jnp.float32)
    g3 = g.reshape(K7 * K7, RT, NPG)
    zc = jnp.zeros((K7 * K7, RT, 3), jnp.float32)
    gp_ref[:, pl.ds(8, 8), :] = jnp.zeros((K7 * K7, 8, NPG + 6), jnp.float32)
    gp_ref[:, pl.ds(80, 8), :] = jnp.zeros((K7 * K7, 8, NPG + 6), jnp.float32)
    gp_ref[:, pl.ds(woff, RT), :] = jnp.concatenate([zc, g3, zc], axis=2)

    y1 = jnp.zeros((TI, NPG), jnp.float32)
    y2 = jnp.zeros((TI, NPG), jnp.float32)
    for a in range(K7):
        for b in range(K7):
            slab = gp_ref[a * K7 + b, pl.ds(13, 70), :]
            y1 = y1 + lax.slice(slab, (a, b), (a + TI, b + NPG))
            y2 = y2 + lax.slice(slab, (b, a), (b + TI, a + NPG))
    bp = bp_ref[0, 0]
    c = 0.5 * (jax.nn.sigmoid(y1 + bp) + jax.nn.sigmoid(y2 + bp))
    out_ref[0, 0] = c


def _decode(h3, h3t, whc, wp49, bh, bp2):
    return pl.pallas_call(
        _decode_body,
        grid=(B, NT),
        in_specs=[
            pl.BlockSpec((NPG, H), lambda b, t: (b, 0)),
            pl.BlockSpec((H, NPG), lambda b, t: (0, b)),
            pl.BlockSpec((H, 2 * H), lambda b, t: (0, 0)),
            pl.BlockSpec((K7 * K7, H), lambda b, t: (0, 0)),
            pl.BlockSpec((H,), lambda b, t: (0,)),
            pl.BlockSpec((1, 1), lambda b, t: (0, 0)),
        ],
        out_specs=pl.BlockSpec((1, 1, TI, NPG), lambda b, t: (b, 0, t, 0)),
        out_shape=jax.ShapeDtypeStruct((B, 1, NPG, NPG), jnp.float32),
        scratch_shapes=[pltpu.VMEM((K7 * K7, GPR, NPG + 6), jnp.float32)],
    )(h3, h3t, whc, wp49, bh, bp2)


def kernel(x, edge_index, edge_attr, batch, W0, b0, W1, b1, W2, b2,
           Wh, bh, Wp, bp):
    src = edge_index[0]
    dst = edge_index[1]
    a = _build_adj(src, dst, edge_attr).reshape(N, N)
    h3, h3t = _gcn(a, x, W0, b0, W1, b1, W2, b2)
    whc = Wh[:, :, 0, 0].astype(jnp.bfloat16)
    wp49 = Wp[0].reshape(H, K7 * K7).T.astype(jnp.bfloat16)
    bp2 = bp.reshape(1, 1)
    return _decode(h3, h3t, whc, wp49, bh, bp2)


# speedup vs baseline: 17.4074x; 1.1985x over previous
import dataclasses
import functools

import numpy as np

import jax
import jax.numpy as jnp
from jax import lax
from jax.experimental import pallas as pl
from jax.experimental.pallas import tpu as pltpu
from jax.experimental.pallas import tpu_sc as plsc

N = 2048
E = 65536
B = 8
NPG = 256
H = 64
K7 = 7
TI = 64
RT = 80
NT = NPG // TI
GPR = 112

_MM_PREC = lax.Precision.DEFAULT


SUB = 16
EPT = E // SUB
SROWS = 512
TRASH = SROWS * N
SHW = TRASH + 128
ZBLK = 16384
SLICE = SROWS * N // SUB


_SC_PARAMS = pltpu.CompilerParams()
if "needs_layout_passes" in pltpu.CompilerParams.__dataclass_fields__:
    _SC_PARAMS = dataclasses.replace(_SC_PARAMS, needs_layout_passes=False)


def _build_adj(src, dst, w):
    mesh = plsc.VectorSubcoreMesh(core_axis_name="c", subcore_axis_name="s")

    @functools.partial(
        pl.kernel,
        out_type=jax.ShapeDtypeStruct((N * N,), jnp.float32),
        mesh=mesh,
        compiler_params=_SC_PARAMS,
        scratch_types=[
            pltpu.VMEM((EPT,), jnp.int32),
            pltpu.VMEM((EPT,), jnp.int32),
            pltpu.VMEM((EPT // 128, 128), jnp.float32),
            pltpu.VMEM((EPT // 128, 128), jnp.int32),
            pltpu.VMEM((ZBLK,), jnp.float32),
            pltpu.VMEM_SHARED((SHW,), jnp.float32),
            pltpu.SemaphoreType.DMA,
        ],
    )
    def k(src_hbm, dst_hbm, w_hbm, out_hbm, s_v, d_v, w2, idx2, zb, shared,
          sem):
        cid = lax.axis_index("c")
        sid = lax.axis_index("s")
        base_e = sid * EPT
        pltpu.sync_copy(src_hbm.at[pl.ds(base_e, EPT)], s_v)
        pltpu.sync_copy(dst_hbm.at[pl.ds(base_e, EPT)], d_v)
        for j in range(EPT // 128):
            pltpu.async_copy(w_hbm.at[pl.ds(base_e + j * 128, 128)],
                             w2.at[j], sem)

        @pl.loop(0, ZBLK // 64)
        def _zb(i):
            z = jnp.zeros((16,), jnp.float32)
            for u in range(4):
                zb[pl.ds(i * 64 + u * 16, 16)] = z

        for j in range(EPT // 128):
            pltpu.make_async_copy(w_hbm.at[pl.ds(base_e + j * 128, 128)],
                                  w2.at[j], sem).wait()

        lanes = lax.iota(jnp.int32, 16)
        for gi in range(2):
            row0 = (cid * 2 + gi) * SROWS
            zoff = pl.multiple_of(sid * SLICE, 64)
            for j in range(SLICE // ZBLK):
                pltpu.async_copy(zb, shared.at[pl.ds(zoff + j * ZBLK, ZBLK)],
                                 sem)
            for j in range(SLICE // ZBLK):
                pltpu.make_async_copy(
                    zb, shared.at[pl.ds(zoff + j * ZBLK, ZBLK)], sem).wait()
            plsc.subcore_barrier()

            @pl.loop(0, EPT // 128)
            def _mkidx(jj):
                for u in range(8):
                    i = jj * 8 + u
                    sl = pl.ds(i * 16, 16)
                    rel = d_v[sl] - row0
                    m = (rel >= 0) & (rel < SROWS)
                    spread = (TRASH + (i & 7) * 16) + lanes
                    idx = jnp.where(m, rel * N + s_v[sl], spread)
                    idx2[jj, pl.ds(u * 16, 16)] = idx

            for j in range(EPT // 128):
                pltpu.sync_copy(w2.at[j], shared.at[idx2.at[j]], add=True)
            plsc.subcore_barrier()

            out0 = pl.multiple_of(row0 * N + sid * SLICE, 64)
            pltpu.sync_copy(shared.at[pl.ds(zoff, SLICE)],
                            out_hbm.at[pl.ds(out0, SLICE)])
            plsc.subcore_barrier()

    return k(src, dst, w)


def _gcn_body(a_ref, x_ref, w0_ref, b0_ref, w1_ref, b1_ref, w2_ref, b2_ref,
              out_ref, outt_ref):
    a = a_ref[...]
    deg = jnp.sum(a, axis=1) + 2.0
    dinv = jnp.where(deg > 0, lax.rsqrt(deg), 0.0)
    dcol = dinv[:, None]
    loop_w = 2.0 * dcol * dcol
    h = x_ref[...]
    for w_r, b_r in ((w0_ref, b0_ref), (w1_ref, b1_ref), (w2_ref, b2_ref)):
        hp = jnp.dot(h, w_r[...], precision=_MM_PREC)
        u = jnp.dot(a, hp * dcol, precision=_MM_PREC)
        h = jax.nn.relu(dcol * u + loop_w * hp + b_r[...][None, :])
    out_ref[...] = h
    outt_ref[...] = h.T.astype(jnp.bfloat16)


def _gcn(a, x, w0, b0, w1, b1, w2, b2):
    return pl.pallas_call(
        _gcn_body,
        out_shape=(jax.ShapeDtypeStruct((N, H), jnp.float32),
                   jax.ShapeDtypeStruct((H, N), jnp.bfloat16)),
    )(a, x, w0, b0, w1, b1, w2, b2)


def _decode_body(z_ref, zt_ref, whc_ref, wp_ref, bh_ref, bp_ref, sel_ref,
                 out_ref, gp_ref):
    t = pl.program_id(1)
    i0 = t * TI
    st = pl.multiple_of(jnp.clip(i0 - 8, 0, NPG - RT), 8)
    off = i0 - st + 3

    zt = zt_ref[...]
    zr = z_ref[pl.ds(st, RT), :]
    zrt = zr.T.astype(jnp.bfloat16)
    diff = jnp.abs(zrt[:, :, None] - zt[:, None, :])
    prod = zrt[:, :, None] * zt[:, None, :]
    cat = jnp.concatenate([diff, prod], axis=0)
    c2 = cat.reshape(2 * H, RT * NPG)
    hmt = jax.nn.relu(
        jnp.dot(whc_ref[...], c2, precision=_MM_PREC,
                preferred_element_type=jnp.float32)
        + bh_ref[...][:, None]).astype(jnp.bfloat16)
    g = jnp.dot(wp_ref[...], hmt, precision=_MM_PREC,
                preferred_element_type=jnp.float32)
    g3 = g.reshape(K7, K7, RT, NPG).astype(jnp.bfloat16)
    zc = jnp.zeros((K7, K7, RT, 3), jnp.bfloat16)
    zs = jnp.zeros((K7, K7, 16, NPG + 6), jnp.bfloat16)
    gp_ref[:, :, pl.ds(0, 16), :] = zs
    gp_ref[:, :, pl.ds(96, 16), :] = zs
    gp_ref[:, :, pl.ds(16, RT), :] = jnp.concatenate([zc, g3, zc], axis=3)

    soff = pl.multiple_of(((off - 3) // 8) * TI, 16)
    sel = sel_ref[pl.ds(soff, TI), :]
    y1 = jnp.zeros((TI, NPG), jnp.float32)
    y2 = jnp.zeros((TI, NPG), jnp.float32)
    for v in range(K7):
        rhs1 = gp_ref[:, v].reshape(K7 * GPR, NPG + 6)
        acc1 = jnp.dot(sel, rhs1, precision=_MM_PREC,
                       preferred_element_type=jnp.float32)
        y1 = y1 + lax.slice(acc1, (0, v), (TI, v + NPG))
        rhs2 = gp_ref[v].reshape(K7 * GPR, NPG + 6)
        acc2 = jnp.dot(sel, rhs2, precision=_MM_PREC,
                       preferred_element_type=jnp.float32)
        y2 = y2 + lax.slice(acc2, (0, v), (TI, v + NPG))
    bp = bp_ref[0, 0]
    c = 0.5 * (jax.nn.sigmoid(y1 + bp) + jax.nn.sigmoid(y2 + bp))
    out_ref[0, 0] = c


_SEL_NP = np.zeros((3 * TI, K7 * GPR), np.float32)
for _ci, _offv in enumerate((3, 11, 19)):
    for _a in range(K7):
        for _l in range(TI):
            _SEL_NP[_ci * TI + _l, _a * GPR + 10 + _offv + _a + _l] = 1.0


def _decode(h3, h3t, whc, wp49, bh, bp2):
    sel = jnp.asarray(_SEL_NP, dtype=jnp.bfloat16)
    return pl.pallas_call(
        _decode_body,
        grid=(B, NT),
        in_specs=[
            pl.BlockSpec((NPG, H), lambda b, t: (b, 0)),
            pl.BlockSpec((H, NPG), lambda b, t: (0, b)),
            pl.BlockSpec((H, 2 * H), lambda b, t: (0, 0)),
            pl.BlockSpec((K7 * K7, H), lambda b, t: (0, 0)),
            pl.BlockSpec((H,), lambda b, t: (0,)),
            pl.BlockSpec((1, 1), lambda b, t: (0, 0)),
            pl.BlockSpec((3 * TI, K7 * GPR), lambda b, t: (0, 0)),
        ],
        out_specs=pl.BlockSpec((1, 1, TI, NPG), lambda b, t: (b, 0, t, 0)),
        out_shape=jax.ShapeDtypeStruct((B, 1, NPG, NPG), jnp.float32),
        scratch_shapes=[pltpu.VMEM((K7, K7, GPR, NPG + 6), jnp.bfloat16)],
    )(h3, h3t, whc, wp49, bh, bp2, sel)


def kernel(x, edge_index, edge_attr, batch, W0, b0, W1, b1, W2, b2,
           Wh, bh, Wp, bp):
    src = edge_index[0]
    dst = edge_index[1]
    a = _build_adj(src, dst, edge_attr).reshape(N, N)
    h3, h3t = _gcn(a, x, W0, b0, W1, b1, W2, b2)
    whc = Wh[:, :, 0, 0].astype(jnp.bfloat16)
    wp49 = Wp[0].reshape(H, K7 * K7).T.astype(jnp.bfloat16)
    bp2 = bp.reshape(1, 1)
    return _decode(h3, h3t, whc, wp49, bh, bp2)


# speedup vs baseline: 19.3551x; 1.1119x over previous
import dataclasses
import functools

import numpy as np

import jax
import jax.numpy as jnp
from jax import lax
from jax.experimental import pallas as pl
from jax.experimental.pallas import tpu as pltpu
from jax.experimental.pallas import tpu_sc as plsc

N = 2048
E = 65536
B = 8
NPG = 256
H = 64
K7 = 7
TI = 128
RT = 144
NT = NPG // TI
GPR = 176

_MM_PREC = lax.Precision.DEFAULT


SUB = 16
EPT = E // SUB
SROWS = 512
TRASH = SROWS * N
SHW = TRASH + 128
ZBLK = 16384
SLICE = SROWS * N // SUB


_SC_PARAMS = pltpu.CompilerParams()
if "needs_layout_passes" in pltpu.CompilerParams.__dataclass_fields__:
    _SC_PARAMS = dataclasses.replace(_SC_PARAMS, needs_layout_passes=False)


def _build_adj(src, dst, w):
    mesh = plsc.VectorSubcoreMesh(core_axis_name="c", subcore_axis_name="s")

    @functools.partial(
        pl.kernel,
        out_type=jax.ShapeDtypeStruct((N * N,), jnp.float32),
        mesh=mesh,
        compiler_params=_SC_PARAMS,
        scratch_types=[
            pltpu.VMEM((EPT,), jnp.int32),
            pltpu.VMEM((EPT,), jnp.int32),
            pltpu.VMEM((EPT // 128, 128), jnp.float32),
            pltpu.VMEM((EPT // 128, 128), jnp.int32),
            pltpu.VMEM((ZBLK,), jnp.float32),
            pltpu.VMEM_SHARED((SHW,), jnp.float32),
            pltpu.SemaphoreType.DMA,
        ],
    )
    def k(src_hbm, dst_hbm, w_hbm, out_hbm, s_v, d_v, w2, idx2, zb, shared,
          sem):
        cid = lax.axis_index("c")
        sid = lax.axis_index("s")
        base_e = sid * EPT
        pltpu.sync_copy(src_hbm.at[pl.ds(base_e, EPT)], s_v)
        pltpu.sync_copy(dst_hbm.at[pl.ds(base_e, EPT)], d_v)
        for j in range(EPT // 128):
            pltpu.async_copy(w_hbm.at[pl.ds(base_e + j * 128, 128)],
                             w2.at[j], sem)

        @pl.loop(0, ZBLK // 64)
        def _zb(i):
            z = jnp.zeros((16,), jnp.float32)
            for u in range(4):
                zb[pl.ds(i * 64 + u * 16, 16)] = z

        for j in range(EPT // 128):
            pltpu.make_async_copy(w_hbm.at[pl.ds(base_e + j * 128, 128)],
                                  w2.at[j], sem).wait()

        lanes = lax.iota(jnp.int32, 16)
        for gi in range(2):
            row0 = (cid * 2 + gi) * SROWS
            zoff = pl.multiple_of(sid * SLICE, 64)
            for j in range(SLICE // ZBLK):
                pltpu.async_copy(zb, shared.at[pl.ds(zoff + j * ZBLK, ZBLK)],
                                 sem)
            for j in range(SLICE // ZBLK):
                pltpu.make_async_copy(
                    zb, shared.at[pl.ds(zoff + j * ZBLK, ZBLK)], sem).wait()
            plsc.subcore_barrier()

            @pl.loop(0, EPT // 128)
            def _mkidx(jj):
                for u in range(8):
                    i = jj * 8 + u
                    sl = pl.ds(i * 16, 16)
                    rel = d_v[sl] - row0
                    m = (rel >= 0) & (rel < SROWS)
                    spread = (TRASH + (i & 7) * 16) + lanes
                    idx = jnp.where(m, rel * N + s_v[sl], spread)
                    idx2[jj, pl.ds(u * 16, 16)] = idx

            for j in range(EPT // 128):
                pltpu.sync_copy(w2.at[j], shared.at[idx2.at[j]], add=True)
            plsc.subcore_barrier()

            out0 = pl.multiple_of(row0 * N + sid * SLICE, 64)
            pltpu.sync_copy(shared.at[pl.ds(zoff, SLICE)],
                            out_hbm.at[pl.ds(out0, SLICE)])
            plsc.subcore_barrier()

    return k(src, dst, w)


def _gcn_body(a_ref, x_ref, w0_ref, b0_ref, w1_ref, b1_ref, w2_ref, b2_ref,
              out_ref, outt_ref):
    a = a_ref[...]
    deg = jnp.sum(a, axis=1) + 2.0
    dinv = jnp.where(deg > 0, lax.rsqrt(deg), 0.0)
    dcol = dinv[:, None]
    loop_w = 2.0 * dcol * dcol
    h = x_ref[...]
    for w_r, b_r in ((w0_ref, b0_ref), (w1_ref, b1_ref), (w2_ref, b2_ref)):
        hp = jnp.dot(h, w_r[...], precision=_MM_PREC)
        u = jnp.dot(a, hp * dcol, precision=_MM_PREC)
        h = jax.nn.relu(dcol * u + loop_w * hp + b_r[...][None, :])
    out_ref[...] = h
    outt_ref[...] = h.T.astype(jnp.bfloat16)


def _gcn(a, x, w0, b0, w1, b1, w2, b2):
    return pl.pallas_call(
        _gcn_body,
        out_shape=(jax.ShapeDtypeStruct((N, H), jnp.float32),
                   jax.ShapeDtypeStruct((H, N), jnp.bfloat16)),
    )(a, x, w0, b0, w1, b1, w2, b2)


def _decode_body(z_ref, zt_ref, whc_ref, wp_ref, bh_ref, bp_ref, sel_ref,
                 out_ref, gp_ref):
    t = pl.program_id(1)
    i0 = t * TI
    st = pl.multiple_of(jnp.clip(i0 - 8, 0, NPG - RT), 8)
    off = i0 - st + 3

    zt = zt_ref[...]
    zr = z_ref[pl.ds(st, RT), :]
    zrt = zr.T.astype(jnp.bfloat16)
    diff = jnp.abs(zrt[:, :, None] - zt[:, None, :])
    prod = zrt[:, :, None] * zt[:, None, :]
    cat = jnp.concatenate([diff, prod], axis=0)
    c2 = cat.reshape(2 * H, RT * NPG)
    hmt = jax.nn.relu(
        jnp.dot(whc_ref[...], c2, precision=_MM_PREC,
                preferred_element_type=jnp.float32)
        + bh_ref[...][:, None]).astype(jnp.bfloat16)
    g = jnp.dot(wp_ref[...], hmt, precision=_MM_PREC,
                preferred_element_type=jnp.float32)
    g3 = g.reshape(K7, K7, RT, NPG).astype(jnp.bfloat16)
    zc = jnp.zeros((K7, K7, RT, 3), jnp.bfloat16)
    zs = jnp.zeros((K7, K7, 16, NPG + 6), jnp.bfloat16)
    gp_ref[:, :, pl.ds(0, 16), :] = zs
    gp_ref[:, :, pl.ds(GPR - 16, 16), :] = zs
    gp_ref[:, :, pl.ds(16, RT), :] = jnp.concatenate([zc, g3, zc], axis=3)

    soff = pl.multiple_of(((off - 3) // 8) * TI, 16)
    sel = sel_ref[pl.ds(soff, TI), :]
    y1 = jnp.zeros((TI, NPG), jnp.float32)
    y2 = jnp.zeros((TI, NPG), jnp.float32)
    for v in range(K7):
        rhs1 = gp_ref[:, v].reshape(K7 * GPR, NPG + 6)
        acc1 = jnp.dot(sel, rhs1, precision=_MM_PREC,
                       preferred_element_type=jnp.float32)
        y1 = y1 + lax.slice(acc1, (0, v), (TI, v + NPG))
        rhs2 = gp_ref[v].reshape(K7 * GPR, NPG + 6)
        acc2 = jnp.dot(sel, rhs2, precision=_MM_PREC,
                       preferred_element_type=jnp.float32)
        y2 = y2 + lax.slice(acc2, (0, v), (TI, v + NPG))
    bp = bp_ref[0, 0]
    c = 0.5 * (jax.nn.sigmoid(y1 + bp) + jax.nn.sigmoid(y2 + bp))
    out_ref[0, 0] = c


_SEL_NP = np.zeros((3 * TI, K7 * GPR), np.float32)
for _ci, _offv in enumerate((3, 11, 19)):
    for _a in range(K7):
        for _l in range(TI):
            _SEL_NP[_ci * TI + _l, _a * GPR + 10 + _offv + _a + _l] = 1.0


def _decode(h3, h3t, whc, wp49, bh, bp2):
    sel = jnp.asarray(_SEL_NP, dtype=jnp.bfloat16)
    return pl.pallas_call(
        _decode_body,
        grid=(B, NT),
        in_specs=[
            pl.BlockSpec((NPG, H), lambda b, t: (b, 0)),
            pl.BlockSpec((H, NPG), lambda b, t: (0, b)),
            pl.BlockSpec((H, 2 * H), lambda b, t: (0, 0)),
            pl.BlockSpec((K7 * K7, H), lambda b, t: (0, 0)),
            pl.BlockSpec((H,), lambda b, t: (0,)),
            pl.BlockSpec((1, 1), lambda b, t: (0, 0)),
            pl.BlockSpec((3 * TI, K7 * GPR), lambda b, t: (0, 0)),
        ],
        out_specs=pl.BlockSpec((1, 1, TI, NPG), lambda b, t: (b, 0, t, 0)),
        out_shape=jax.ShapeDtypeStruct((B, 1, NPG, NPG), jnp.float32),
        scratch_shapes=[pltpu.VMEM((K7, K7, GPR, NPG + 6), jnp.bfloat16)],
    )(h3, h3t, whc, wp49, bh, bp2, sel)


def kernel(x, edge_index, edge_attr, batch, W0, b0, W1, b1, W2, b2,
           Wh, bh, Wp, bp):
    src = edge_index[0]
    dst = edge_index[1]
    a = _build_adj(src, dst, edge_attr).reshape(N, N)
    h3, h3t = _gcn(a, x, W0, b0, W1, b1, W2, b2)
    whc = Wh[:, :, 0, 0].astype(jnp.bfloat16)
    wp49 = Wp[0].reshape(H, K7 * K7).T.astype(jnp.bfloat16)
    bp2 = bp.reshape(1, 1)
    return _decode(h3, h3t, whc, wp49, bh, bp2)
